# baseline (speedup 1.0000x reference)
import functools
import math

import jax
import jax.numpy as jnp
import numpy as np
from jax import lax
from jax.experimental import pallas as pl
from jax.experimental.pallas import tpu as pltpu
from jax.experimental.pallas import tpu_sc as plsc

D_MODEL = 1024
LANES = 16
NUM_CORES = 2
NUM_SUBCORES = 16
NUM_WORKERS = NUM_CORES * NUM_SUBCORES
B_TOTAL = 32768

T_TC = 12288
B_SC = B_TOTAL - T_TC
B_PER_W = B_SC // NUM_WORKERS
CHUNK = 8
NX = 8
NP = 4
LOOKAHEAD = 3
N_CHUNKS = B_PER_W // CHUNK
N_SUPER = N_CHUNKS // NX
N_SLICES = CHUNK * (D_MODEL // LANES)

TBLK = 512


def _sc_body(x_hbm, pos_hbm, pe_hbm, out_hbm,
             idx_all, pe_v, x_v, gat_sem, xin_sem, out_sem):
    wid = lax.axis_index("s") * NUM_CORES + lax.axis_index("c")
    base = T_TC + wid * B_PER_W

    pltpu.sync_copy(pos_hbm.at[pl.ds(base, B_PER_W)], idx_all)

    def idx_ref(c):
        return idx_all.at[pl.ds(c * CHUNK, CHUNK)]

    def rows(c):
        return pl.ds(base + c * CHUNK, CHUNK)

    def issue_loads(c, sx, sp):
        pltpu.async_copy(pe_hbm.at[idx_ref(c)], pe_v.at[sp], gat_sem.at[sp])
        pltpu.async_copy(x_hbm.at[rows(c)], x_v.at[sx], xin_sem.at[sx])

    for c in range(LOOKAHEAD):
        issue_loads(c, c, c)

    def super_step(g, _):
        for b in range(NX):
            c = g * NX + b
            sx, sp = b, b % NP
            cl = c + LOOKAHEAD
            sx_l, sp_l = (b + LOOKAHEAD) % NX, (b + LOOKAHEAD) % NP

            @pl.when(cl < N_CHUNKS)
            def _():
                @pl.when(cl >= NX)
                def _():
                    pltpu.make_async_copy(
                        x_v.at[sx_l], out_hbm.at[rows(cl - NX)],
                        out_sem.at[sx_l]).wait()
                issue_loads(cl, sx_l, sp_l)

            pltpu.make_async_copy(
                pe_hbm.at[idx_ref(c)], pe_v.at[sp], gat_sem.at[sp]).wait()
            pltpu.make_async_copy(
                x_hbm.at[rows(c)], x_v.at[sx], xin_sem.at[sx]).wait()

            n_sl = D_MODEL // LANES

            @plsc.parallel_loop(0, N_SLICES, 1, unroll=8)
            def _(i):
                t = i // n_sl
                j = i - t * n_sl
                d = pl.ds(j * LANES, LANES)
                plsc.addupdate(x_v.at[sx, t, d], pe_v[sp, t, d])

            pltpu.async_copy(x_v.at[sx], out_hbm.at[rows(c)], out_sem.at[sx])
        return 0

    lax.fori_loop(0, N_SUPER, super_step, 0)

    for b in range(NX):
        c = N_CHUNKS - NX + b
        pltpu.make_async_copy(
            x_v.at[b], out_hbm.at[rows(c)], out_sem.at[b]).wait()


_INV_PI = 0.3183098861837907
_PI1 = 3.140625
_PI2 = float(np.float32(math.pi - 3.140625))
_PI3 = math.pi - 3.140625 - _PI2


def _tc_body(posf_ref, tbl_ref, x_ref, o_ref):
    ang = posf_ref[:] * tbl_ref[0:1, :] + tbl_ref[1:2, :]
    kf = jnp.floor(ang * _INV_PI + 0.5)
    r = ((ang - kf * _PI1) - kf * _PI2) - kf * _PI3
    ki = kf.astype(jnp.int32)
    sgn = jnp.where((ki & 1) == 0, 1.0, -1.0)
    r2 = r * r
    p = r * (1.0 + r2 * (-1.0 / 6.0 + r2 * (1.0 / 120.0 + r2 * (
        -1.0 / 5040.0 + r2 * (1.0 / 362880.0)))))
    o_ref[:] = x_ref[:] + sgn * p


@jax.jit
def _pe_add(x2d, pos1d, pe):
    mesh = plsc.VectorSubcoreMesh(core_axis_name="c", subcore_axis_name="s")
    sc_kern = functools.partial(
        pl.kernel,
        mesh=mesh,
        out_type=jax.ShapeDtypeStruct((B_TOTAL, D_MODEL), jnp.float32),
        scratch_types=[
            pltpu.VMEM((B_PER_W,), jnp.int32),
            pltpu.VMEM((NP, CHUNK, D_MODEL), jnp.float32),
            pltpu.VMEM((NX, CHUNK, D_MODEL), jnp.float32),
            pltpu.SemaphoreType.DMA((NP,)),
            pltpu.SemaphoreType.DMA((NX,)),
            pltpu.SemaphoreType.DMA((NX,)),
        ],
    )(_sc_body)
    o_sc = sc_kern(x2d, pos1d, pe)

    half = jnp.exp(jnp.arange(0, D_MODEL, 2, dtype=jnp.float32)
                   * (-math.log(10000.0) / D_MODEL))
    freq = jnp.repeat(half, 2)
    phase = jnp.tile(jnp.array([0.0, math.pi / 2], dtype=jnp.float32),
                     D_MODEL // 2)
    tbl = jnp.stack([freq, phase])
    posf = pos1d.astype(jnp.float32)[:, None]

    o_tc = pl.pallas_call(
        _tc_body,
        grid=(T_TC // TBLK,),
        in_specs=[
            pl.BlockSpec((TBLK, 1), lambda i: (i, 0)),
            pl.BlockSpec((2, D_MODEL), lambda i: (0, 0)),
            pl.BlockSpec((TBLK, D_MODEL), lambda i: (i, 0)),
        ],
        out_specs=pl.BlockSpec((TBLK, D_MODEL), lambda i: (i, 0)),
        out_shape=jax.ShapeDtypeStruct((T_TC, D_MODEL), jnp.float32),
    )(posf, tbl, x2d)

    return lax.dynamic_update_slice(o_sc, o_tc, (0, 0))


def kernel(x, position, pe):
    b, s, d = x.shape
    x2d = x.reshape(b * s, d)
    pos1d = position.reshape(b * s).astype(jnp.int32)
    out = _pe_add(x2d, pos1d, pe)
    return out.reshape(b, s, d)


# speedup vs baseline: 1.1471x; 1.1471x over previous
import functools

import jax
import jax.numpy as jnp
from jax import lax
from jax.experimental import pallas as pl
from jax.experimental.pallas import tpu as pltpu
from jax.experimental.pallas import tpu_sc as plsc

D_MODEL = 1024
LANES = 16
NUM_CORES = 2
NUM_SUBCORES = 16
NUM_WORKERS = NUM_CORES * NUM_SUBCORES
B_TOTAL = 32768
B_PER_W = B_TOTAL // NUM_WORKERS
CHUNK = 8
NBUF = 4
LOOKAHEAD = 2
N_CHUNKS = B_PER_W // CHUNK
N_SUPER = N_CHUNKS // NBUF
N_SLICES = CHUNK * (D_MODEL // LANES)


def _sc_body(x_hbm, pos_hbm, pe_hbm, out_hbm,
             idx_all, pe_v, x_v, gat_sem, xin_sem, out_sem):
    wid = lax.axis_index("s") * NUM_CORES + lax.axis_index("c")
    base = wid * B_PER_W

    pltpu.sync_copy(pos_hbm.at[pl.ds(base, B_PER_W)], idx_all)

    def idx_ref(c):
        return idx_all.at[pl.ds(c * CHUNK, CHUNK)]

    def rows(c):
        return pl.ds(base + c * CHUNK, CHUNK)

    def issue_loads(c, s):
        pltpu.async_copy(pe_hbm.at[idx_ref(c)], pe_v.at[s], gat_sem.at[s])
        pltpu.async_copy(x_hbm.at[rows(c)], x_v.at[s], xin_sem.at[s])

    for c in range(LOOKAHEAD):
        issue_loads(c, c)

    def super_step(g, _):
        for b in range(NBUF):
            c = g * NBUF + b
            cl = c + LOOKAHEAD
            sl = (b + LOOKAHEAD) % NBUF

            @pl.when(cl < N_CHUNKS)
            def _():
                @pl.when(cl >= NBUF)
                def _():
                    pltpu.make_async_copy(
                        x_v.at[sl], out_hbm.at[rows(cl - NBUF)],
                        out_sem.at[sl]).wait()
                issue_loads(cl, sl)

            pltpu.make_async_copy(
                pe_hbm.at[idx_ref(c)], pe_v.at[b], gat_sem.at[b]).wait()
            pltpu.make_async_copy(
                x_hbm.at[rows(c)], x_v.at[b], xin_sem.at[b]).wait()

            n_sl = D_MODEL // LANES

            @plsc.parallel_loop(0, N_SLICES, 1, unroll=8)
            def _(i):
                t = i // n_sl
                j = i - t * n_sl
                d = pl.ds(j * LANES, LANES)
                plsc.addupdate(x_v.at[b, t, d], pe_v[b, t, d])

            pltpu.async_copy(x_v.at[b], out_hbm.at[rows(c)], out_sem.at[b])
        return 0

    lax.fori_loop(0, N_SUPER, super_step, 0)

    for b in range(NBUF):
        c = N_CHUNKS - NBUF + b
        pltpu.make_async_copy(
            x_v.at[b], out_hbm.at[rows(c)], out_sem.at[b]).wait()


@jax.jit
def _pe_add(x2d, pos1d, pe):
    mesh = plsc.VectorSubcoreMesh(core_axis_name="c", subcore_axis_name="s")
    kern = functools.partial(
        pl.kernel,
        mesh=mesh,
        out_type=jax.ShapeDtypeStruct((B_TOTAL, D_MODEL), jnp.float32),
        scratch_types=[
            pltpu.VMEM((B_PER_W,), jnp.int32),
            pltpu.VMEM((NBUF, CHUNK, D_MODEL), jnp.float32),
            pltpu.VMEM((NBUF, CHUNK, D_MODEL), jnp.float32),
            pltpu.SemaphoreType.DMA((NBUF,)),
            pltpu.SemaphoreType.DMA((NBUF,)),
            pltpu.SemaphoreType.DMA((NBUF,)),
        ],
    )(_sc_body)
    return kern(x2d, pos1d, pe)


def kernel(x, position, pe):
    b, s, d = x.shape
    x2d = x.reshape(b * s, d)
    pos1d = position.reshape(b * s).astype(jnp.int32)
    out = _pe_add(x2d, pos1d, pe)
    return out.reshape(b, s, d)
